# baseline (device time: 119230 ns/iter reference)
import jax
import jax.numpy as jnp
from jax import lax
from jax.experimental import pallas as pl
from jax.experimental.pallas import tpu as pltpu

N_DEV = 32
NZ = 4
NS = 8
WIRE = jnp.bfloat16
F32 = jnp.float32


def _fused_post_attn(attn_partial, x0, mods, W_ff1, W_ff2):
    m, n = attn_partial.shape
    chunk = m // N_DEV
    half = m // 2
    eps = 1e-5
    ap_w = attn_partial.astype(WIRE)

    def body(ap_ref, x0_ref, mods_ref, w1_ref, w2_ref, out_ref,
             st1, l1a, z1a, l2a, g1a, g2a, st2, l1b, z1b, l2b, g1b, g2b,
             x1_buf, sems, bar_sems):
        my = lax.axis_index("i")
        my_z = lax.div(my, NS)
        my_s = lax.rem(my, NS)
        my_lo = my * chunk

        gbar = pltpu.get_barrier_semaphore()
        for r in range(5):
            peer = lax.rem(my + (1 << r), N_DEV)
            sem = gbar if r == 0 else bar_sems.at[r]
            pl.semaphore_signal(sem, inc=1, device_id=(peer,),
                                device_id_type=pl.DeviceIdType.MESH)
            pl.semaphore_wait(sem, 1)

        def rdma(src_ref, dst_ref, send_sem, recv_sem, dev):
            return pltpu.make_async_remote_copy(
                src_ref=src_ref, dst_ref=dst_ref, send_sem=send_sem,
                recv_sem=recv_sem, device_id=(dev,),
                device_id_type=pl.DeviceIdType.MESH)

        drains = []

        def all_reduce(src_w, st, l1, zst, l2, g1, g2, s_off):
            s1s, s1r = sems.at[s_off + 0], sems.at[s_off + 1]
            s2s, s2r = sems.at[s_off + 2], sems.at[s_off + 3]
            g1s, g1r = sems.at[s_off + 4], sems.at[s_off + 5]
            g2s, g2r = sems.at[s_off + 6], sems.at[s_off + 7]

            st[...] = jnp.swapaxes(src_w.reshape(NZ, NS, chunk, n), 0, 1)

            for j in range(NS):
                @pl.when(my_s != j)
                def _(j=j):
                    rdma(st.at[j], l1.at[my_s], s1s.at[j], s1r.at[my_s],
                         my_z * NS + j).start()
            l1[my_s] = st[my_s]
            for j in range(NS):
                @pl.when(my_s != j)
                def _(j=j):
                    rdma(l1.at[j], l1.at[j], s1s.at[j], s1r.at[j],
                         my).wait_recv()
            blk = jnp.sum(l1[...].astype(F32), axis=0)
            zst[...] = blk.astype(WIRE)

            for k in range(NZ):
                @pl.when(my_z != k)
                def _(k=k):
                    rdma(zst.at[k], l2.at[my_z], s2s.at[k], s2r.at[my_z],
                         k * NS + my_s).start()
            l2[my_z] = zst[my_z]
            for k in range(NZ):
                @pl.when(my_z != k)
                def _(k=k):
                    rdma(l2.at[k], l2.at[k], s2s.at[k], s2r.at[k],
                         my).wait_recv()
            acc = jnp.sum(l2[...].astype(F32), axis=0)

            g1[my_z] = acc.astype(WIRE)
            for k in range(NZ):
                @pl.when(my_z != k)
                def _(k=k):
                    rdma(g1.at[my_z], g1.at[my_z], g1s.at[k], g1r.at[my_z],
                         k * NS + my_s).start()
            for k in range(NZ):
                @pl.when(my_z != k)
                def _(k=k):
                    rdma(g1.at[k], g1.at[k], g1s.at[k], g1r.at[k],
                         my).wait_recv()

            g2[my_s] = g1[...]
            for j in range(NS):
                @pl.when(my_s != j)
                def _(j=j):
                    rdma(g2.at[my_s], g2.at[my_s], g2s.at[j], g2r.at[my_s],
                         my_z * NS + j).start()
            for j in range(NS):
                @pl.when(my_s != j)
                def _(j=j):
                    rdma(g2.at[j], g2.at[j], g2s.at[j], g2r.at[j],
                         my).wait_recv()
            full = jnp.swapaxes(g2[...], 0, 1).reshape(m, n).astype(F32)
            drains.extend([(st, s1s, NS, "s"), (zst, s2s, NZ, "z"),
                           (g1, g1s, NZ, "z"), (g2, g2s, NS, "s")])
            return acc, full

        acc1, a1 = all_reduce(ap_ref[...], st1, l1a, z1a, l2a, g1a, g2a, 0)

        ffp_halves = []
        for b in range(2):
            lo = b * half
            ga = mods_ref[b:b + 1, :]
            sm = mods_ref[2 + b:3 + b, :]
            shm = mods_ref[4 + b:5 + b, :]
            x1 = x0_ref[lo:lo + half, :] + ga * a1[lo:lo + half]
            x1_buf[lo:lo + half, :] = x1
            mu = jnp.mean(x1, axis=-1, keepdims=True)
            var = jnp.mean((x1 - mu) * (x1 - mu), axis=-1, keepdims=True)
            xm = ((x1 - mu) / jnp.sqrt(var + eps)) * (1.0 + sm) + shm
            h = jnp.dot(xm, w1_ref[...], preferred_element_type=F32)
            h = h * (1.0 / (1.0 + jnp.exp(-h)))
            ffp_halves.append(jnp.dot(h, w2_ref[...],
                                      preferred_element_type=F32))
        ffp = jnp.concatenate(ffp_halves, axis=0).astype(WIRE)

        acc2, ff = all_reduce(ffp, st2, l1b, z1b, l2b, g1b, g2b, 8)

        out_ref[0:half, :] = x1_buf[0:half, :] + mods_ref[6:7, :] * ff[0:half]
        out_ref[half:m, :] = x1_buf[half:m, :] + mods_ref[7:8, :] * ff[half:m]
        gm_my = jnp.where(my < N_DEV // 2, mods_ref[6:7, :], mods_ref[7:8, :])
        out_ref[pl.ds(my_lo, chunk), :] = (
            x1_buf[pl.ds(my_lo, chunk), :] + gm_my * acc2)

        for buf, sem, cnt, kind in drains:
            for i in range(cnt):
                cond = (my_s != i) if kind == "s" else (my_z != i)
                @pl.when(cond)
                def _(buf=buf, sem=sem, i=i):
                    rdma(buf.at[0], buf.at[0], sem.at[i], sem.at[i],
                         my).wait_send()

    return pl.pallas_call(
        body,
        out_shape=jax.ShapeDtypeStruct((m, n), F32),
        in_specs=[pl.BlockSpec(memory_space=pltpu.VMEM)] * 5,
        out_specs=pl.BlockSpec(memory_space=pltpu.VMEM),
        scratch_shapes=[
            pltpu.VMEM((NS, NZ, chunk, n), WIRE),
            pltpu.VMEM((NS, NZ, chunk, n), WIRE),
            pltpu.VMEM((NZ, chunk, n), WIRE),
            pltpu.VMEM((NZ, chunk, n), WIRE),
            pltpu.VMEM((NZ, chunk, n), WIRE),
            pltpu.VMEM((NS, NZ, chunk, n), WIRE),
            pltpu.VMEM((NS, NZ, chunk, n), WIRE),
            pltpu.VMEM((NS, NZ, chunk, n), WIRE),
            pltpu.VMEM((NZ, chunk, n), WIRE),
            pltpu.VMEM((NZ, chunk, n), WIRE),
            pltpu.VMEM((NZ, chunk, n), WIRE),
            pltpu.VMEM((NS, NZ, chunk, n), WIRE),
            pltpu.VMEM((m, n), F32),
            pltpu.SemaphoreType.DMA((16, NS)),
            pltpu.SemaphoreType.REGULAR((5,)),
        ],
        compiler_params=pltpu.CompilerParams(collective_id=0),
    )(ap_w, x0, mods, W_ff1, W_ff2)


def kernel(x, Wq, Wk, Wv, Wo, t_emb, W_mod, W_ff1, W_ff2):
    B, S, D = x.shape
    eps = 1e-5
    Dh = 96
    Hq = Wq.shape[1] // Dh

    mod = t_emb @ W_mod
    sa, sha, ga, sm, shm, gm = jnp.split(mod, 6, axis=-1)

    x0 = x
    mu = x0.mean(axis=-1, keepdims=True)
    var = x0.var(axis=-1, keepdims=True)
    xa = ((x0 - mu) / jnp.sqrt(var + eps)) * (1.0 + sa[:, None, :]) + sha[:, None, :]

    Q = (xa @ Wq).reshape(B, S, Hq, Dh)
    K = (xa @ Wk).reshape(B, S, Hq, Dh)
    V = (xa @ Wv).reshape(B, S, Hq, Dh)
    scores = jnp.einsum("bihd,bjhd->bhij", Q, K) * 0.10206207261596577
    p = jax.nn.softmax(scores, axis=-1)
    attn = jnp.einsum("bhij,bjhd->bihd", p, V).reshape(B, S, Hq * Dh)
    attn_partial = attn @ Wo

    mods = jnp.concatenate([ga, sm, shm, gm], axis=0)
    out = _fused_post_attn(
        attn_partial.reshape(B * S, D), x0.reshape(B * S, D), mods, W_ff1, W_ff2
    )
    return out.reshape(B, S, D)
